# baseline (speedup 1.0000x reference)
import functools

import jax
import jax.numpy as jnp
from jax import lax
from jax.experimental import pallas as pl
from jax.experimental.pallas import tpu as pltpu
from jax.experimental.pallas import tpu_sc as plsc

D = 32
B = 4096
NC, NS, L = 2, 16, 16
NW = NC * NS
BPW = B // NW
GROUPS = BPW // L

_sc_mesh = plsc.VectorSubcoreMesh(core_axis_name="c", subcore_axis_name="s")


@functools.partial(
    pl.kernel,
    out_type=(
        jax.ShapeDtypeStruct((B, D), jnp.float32),
        jax.ShapeDtypeStruct((B, D), jnp.float32),
        jax.ShapeDtypeStruct((B,), jnp.float32),
    ),
    mesh=_sc_mesh,
    compiler_params=pltpu.CompilerParams(use_tc_tiling_on_sc=False),
    scratch_types=[
        pltpu.VMEM((BPW,), jnp.int32),
        pltpu.VMEM((BPW,), jnp.int32),
        pltpu.VMEM((BPW, D), jnp.float32),
        pltpu.VMEM((BPW, D), jnp.float32),
        pltpu.VMEM((BPW,), jnp.float32),
        pltpu.VMEM((BPW,), jnp.float32),
        pltpu.SemaphoreType.DMA,
        pltpu.SemaphoreType.DMA,
        pltpu.SemaphoreType.DMA,
        pltpu.SemaphoreType.DMA,
    ],
)
def _sc_gather(uemb, iemb, uids, iids, ubias, ibias,
               ue_out, ie_out, brow_out,
               uid_v, iid_v, urows, irows, ubv, ibv,
               sem_u, sem_i, sem_ub, sem_ib):
    wid = lax.axis_index("s") * NC + lax.axis_index("c")
    base = wid * BPW
    pltpu.sync_copy(uids.at[pl.ds(base, BPW)], uid_v)
    pltpu.sync_copy(iids.at[pl.ds(base, BPW)], iid_v)
    cu = pltpu.async_copy(uemb.at[uid_v], urows, sem_u)
    ci = pltpu.async_copy(iemb.at[iid_v], irows, sem_i)
    cub = pltpu.async_copy(ubias.at[uid_v], ubv, sem_ub)
    cib = pltpu.async_copy(ibias.at[iid_v], ibv, sem_ib)
    cu.wait()
    pltpu.sync_copy(urows, ue_out.at[pl.ds(base, BPW)])
    ci.wait()
    pltpu.sync_copy(irows, ie_out.at[pl.ds(base, BPW)])
    cub.wait()
    cib.wait()
    for g in range(GROUPS):
        s = pl.ds(g * L, L)
        ubv[s] = ubv[s] + ibv[s]
    pltpu.sync_copy(ubv, brow_out.at[pl.ds(base, BPW)])


def _dot_body(ue_ref, ie_ref, o_ref):
    o_ref[...] = jnp.sum(ue_ref[...] * ie_ref[...], axis=1, keepdims=True)


def _bcast_body(dot_ref, brow_ref, out_ref):
    out_ref[...] = brow_ref[...] + dot_ref[...]


TILE_I = 512


@jax.jit
def _tc_stage(ue, ie, brow):
    dot_col = pl.pallas_call(
        _dot_body,
        out_shape=jax.ShapeDtypeStruct((B, 1), jnp.float32),
    )(ue, ie)
    return pl.pallas_call(
        _bcast_body,
        grid=(B // TILE_I,),
        in_specs=[
            pl.BlockSpec((1, B), lambda i: (0, 0)),
            pl.BlockSpec((TILE_I, 1), lambda i: (i, 0)),
        ],
        out_specs=pl.BlockSpec((TILE_I, B), lambda i: (i, 0)),
        out_shape=jax.ShapeDtypeStruct((B, B), jnp.float32),
    )(dot_col.reshape(1, B), brow.reshape(B, 1))


def kernel(user_ids, item_ids, user_emb, item_emb, user_bias, item_bias):
    uids = user_ids.astype(jnp.int32)
    iids = item_ids.astype(jnp.int32)
    ue, ie, brow = _sc_gather(
        user_emb, item_emb, uids, iids,
        user_bias.reshape(-1), item_bias.reshape(-1))
    return _tc_stage(ue, ie, brow)
